# baseline (speedup 1.0000x reference)
import jax
import jax.numpy as jnp
from jax import lax
from jax.experimental import pallas as pl
from jax.experimental.pallas import tpu as pltpu, tpu_sc as plsc

NC, NS = 2, 16
NW = NC * NS
B, L, D = 16384, 200, 32
V = 1000000
BPW = B // NW
G = 8
NG = BPW // G
UN = 8
C0 = 128
C1 = L - C0
BK = 1024


def _tbody(t_ref, out_ref):
    blk = t_ref[...]
    r = lax.broadcasted_iota(jnp.int32, (BK, BK // 4), 0)
    m = lax.broadcasted_iota(jnp.int32, (BK, BK // 4), 1)
    qs = []
    for a in range(4):
        sel = (r == 4 * m + a).astype(jnp.float32)
        qs.append(jnp.dot(blk, sel, preferred_element_type=jnp.float32,
                          precision=lax.Precision.HIGHEST))
    cat = jnp.concatenate(qs, axis=0)
    out_ref[...] = cat.T


def _repack_table(table):
    tT = table.T
    nblk = (V + BK - 1) // BK
    packed = pl.pallas_call(
        _tbody,
        grid=(nblk,),
        in_specs=[pl.BlockSpec((D, BK), lambda i: (0, i))],
        out_specs=pl.BlockSpec((BK // 4, 128), lambda i: (i, 0)),
        out_shape=jax.ShapeDtypeStruct((V // 4, 128), jnp.float32),
    )(tT)
    return packed.reshape(V, D)


def _body(d0_hbm, d1_hbm, table_hbm, out_hbm, idx0_v, idx1_v, rows_v, out_v,
          sem0, sem1):
    wid = lax.axis_index("s") * NC + lax.axis_index("c")
    base_row = wid * BPW
    sems = (sem0, sem1)

    def fire(g, b):
        row0 = base_row + g * G
        pltpu.sync_copy(d0_hbm.at[pl.ds(row0, G)], idx0_v.at[b])
        pltpu.sync_copy(d1_hbm.at[pl.ds(row0, G)], idx1_v.at[b])
        for r in range(G):
            pltpu.async_copy(
                table_hbm.at[idx0_v.at[b, r]],
                rows_v.at[b, pl.ds(r * L, C0)],
                sems[b],
            )
            pltpu.async_copy(
                table_hbm.at[idx1_v.at[b, r, pl.ds(128 - C1, C1)]],
                rows_v.at[b, pl.ds(r * L + C0, C1)],
                sems[b],
            )

    def drain(b):
        pltpu.make_async_copy(
            table_hbm.at[pl.ds(0, G * L)], rows_v.at[b], sems[b]
        ).wait()

    def accum(g, b):
        for r in range(G):
            e0 = r * L

            def rbody(i, accs):
                a0, a1 = accs
                e = e0 + i * UN
                for k in range(UN):
                    a0 = a0 + rows_v[b, e + k, 0:16]
                    a1 = a1 + rows_v[b, e + k, 16:32]
                return a0, a1

            z = jnp.zeros((16,), jnp.float32)
            a0, a1 = lax.fori_loop(0, L // UN, rbody, (z, z))
            out_v[b, r, 0:16] = a0
            out_v[b, r, 16:32] = a1
        pltpu.sync_copy(out_v.at[b], out_hbm.at[pl.ds(base_row + g * G, G)])

    fire(0, 0)

    @pl.loop(0, NG - 2, step=2)
    def _(g):
        fire(g + 1, 1)
        drain(0)
        accum(g, 0)
        fire(g + 2, 0)
        drain(1)
        accum(g + 1, 1)

    fire(NG - 1, 1)
    drain(0)
    accum(NG - 2, 0)
    drain(1)
    accum(NG - 1, 1)


def kernel(data, lengths, table):
    del lengths
    d0 = lax.slice(data, (0, 0), (B, C0))
    d1 = lax.slice(data, (0, L - 128), (B, L))
    table_c = _repack_table(table)
    mesh = plsc.VectorSubcoreMesh(core_axis_name="c", subcore_axis_name="s")
    f = pl.kernel(
        _body,
        out_type=jax.ShapeDtypeStruct((B, D), jnp.float32),
        mesh=mesh,
        scratch_types=[
            pltpu.VMEM((2, G, 128), jnp.int32),
            pltpu.VMEM((2, G, 128), jnp.int32),
            pltpu.VMEM((2, G * L, D), jnp.float32),
            pltpu.VMEM((2, G, D), jnp.float32),
            pltpu.SemaphoreType.DMA,
            pltpu.SemaphoreType.DMA,
        ],
        compiler_params=pltpu.CompilerParams(use_tc_tiling_on_sc=False),
    )
    return f(d0, d1, table_c)


# speedup vs baseline: 1.0055x; 1.0055x over previous
import jax
import jax.numpy as jnp
from jax import lax
from jax.experimental import pallas as pl
from jax.experimental.pallas import tpu as pltpu, tpu_sc as plsc

NC, NS = 2, 16
NW = NC * NS
B, L, D = 16384, 200, 32
V = 1000000
BPW = B // NW
G = 8
NG = BPW // G
UN = 8
C0 = 128
C1 = L - C0
BK = 256


def _tbody(t_ref, out_ref, sel_ref):
    @pl.when(pl.program_id(0) == 0)
    def _():
        r = lax.broadcasted_iota(jnp.int32, (4, BK, BK // 4), 1)
        m = lax.broadcasted_iota(jnp.int32, (4, BK, BK // 4), 2)
        a = lax.broadcasted_iota(jnp.int32, (4, BK, BK // 4), 0)
        sel_ref[...] = (r == 4 * m + a).astype(jnp.float32)

    blk = t_ref[...]
    qs = []
    for a in range(4):
        qs.append(jnp.dot(blk, sel_ref[a], preferred_element_type=jnp.float32,
                          precision=lax.Precision.HIGHEST))
    cat = jnp.concatenate(qs, axis=0)
    out_ref[...] = cat.T


def _repack_table(table):
    tT = table.T
    nblk = (V + BK - 1) // BK
    packed = pl.pallas_call(
        _tbody,
        grid=(nblk,),
        in_specs=[pl.BlockSpec((D, BK), lambda i: (0, i))],
        out_specs=pl.BlockSpec((BK // 4, 128), lambda i: (i, 0)),
        out_shape=jax.ShapeDtypeStruct((V // 4, 128), jnp.float32),
        scratch_shapes=[pltpu.VMEM((4, BK, BK // 4), jnp.float32)],
    )(tT)
    return packed.reshape(V, D)


def _body(d0_hbm, d1_hbm, table_hbm, out_hbm, idx0_v, idx1_v, rows_v, out_v,
          sem0, sem1):
    wid = lax.axis_index("s") * NC + lax.axis_index("c")
    base_row = wid * BPW
    sems = (sem0, sem1)

    def fire(g, b):
        row0 = base_row + g * G
        pltpu.sync_copy(d0_hbm.at[pl.ds(row0, G)], idx0_v.at[b])
        pltpu.sync_copy(d1_hbm.at[pl.ds(row0, G)], idx1_v.at[b])
        for r in range(G):
            pltpu.async_copy(
                table_hbm.at[idx0_v.at[b, r]],
                rows_v.at[b, pl.ds(r * L, C0)],
                sems[b],
            )
            pltpu.async_copy(
                table_hbm.at[idx1_v.at[b, r, pl.ds(128 - C1, C1)]],
                rows_v.at[b, pl.ds(r * L + C0, C1)],
                sems[b],
            )

    def drain(b):
        pltpu.make_async_copy(
            table_hbm.at[pl.ds(0, G * L)], rows_v.at[b], sems[b]
        ).wait()

    def accum(g, b):
        for r in range(G):
            e0 = r * L

            def rbody(i, accs):
                a0, a1 = accs
                e = e0 + i * UN
                for k in range(UN):
                    a0 = a0 + rows_v[b, e + k, 0:16]
                    a1 = a1 + rows_v[b, e + k, 16:32]
                return a0, a1

            z = jnp.zeros((16,), jnp.float32)
            a0, a1 = lax.fori_loop(0, L // UN, rbody, (z, z))
            out_v[b, r, 0:16] = a0
            out_v[b, r, 16:32] = a1
        pltpu.sync_copy(out_v.at[b], out_hbm.at[pl.ds(base_row + g * G, G)])

    fire(0, 0)

    @pl.loop(0, NG - 2, step=2)
    def _(g):
        fire(g + 1, 1)
        drain(0)
        accum(g, 0)
        fire(g + 2, 0)
        drain(1)
        accum(g + 1, 1)

    fire(NG - 1, 1)
    drain(0)
    accum(NG - 2, 0)
    drain(1)
    accum(NG - 1, 1)


def kernel(data, lengths, table):
    del lengths
    d0 = lax.slice(data, (0, 0), (B, C0))
    d1 = lax.slice(data, (0, L - 128), (B, L))
    table_c = _repack_table(table)
    mesh = plsc.VectorSubcoreMesh(core_axis_name="c", subcore_axis_name="s")
    f = pl.kernel(
        _body,
        out_type=jax.ShapeDtypeStruct((B, D), jnp.float32),
        mesh=mesh,
        scratch_types=[
            pltpu.VMEM((2, G, 128), jnp.int32),
            pltpu.VMEM((2, G, 128), jnp.int32),
            pltpu.VMEM((2, G * L, D), jnp.float32),
            pltpu.VMEM((2, G, D), jnp.float32),
            pltpu.SemaphoreType.DMA,
            pltpu.SemaphoreType.DMA,
        ],
        compiler_params=pltpu.CompilerParams(use_tc_tiling_on_sc=False),
    )
    return f(d0, d1, table_c)


# speedup vs baseline: 4.2422x; 4.2191x over previous
import jax
import jax.numpy as jnp
from jax import lax
from jax.experimental import pallas as pl
from jax.experimental.pallas import tpu as pltpu, tpu_sc as plsc

NC, NS = 2, 16
NW = NC * NS
B, L, D = 16384, 200, 32
V = 1000000
BPW = B // NW
G = 8
NG = BPW // G
UN = 8
C0 = 128
C1 = L - C0
BK = 8192


def _tbody(t_ref, out_ref):
    full = t_ref[...].T
    g = full.reshape(BK // 4, 4, D)
    out_ref[...] = jnp.concatenate([g[:, a, :] for a in range(4)], axis=1)


def _repack_table(table):
    tT = table.T
    nblk = (V + BK - 1) // BK
    packed = pl.pallas_call(
        _tbody,
        grid=(nblk,),
        in_specs=[pl.BlockSpec((D, BK), lambda i: (0, i))],
        out_specs=pl.BlockSpec((BK // 4, 128), lambda i: (i, 0)),
        out_shape=jax.ShapeDtypeStruct((V // 4, 128), jnp.float32),
    )(tT)
    return packed.reshape(V, D)


def _body(d0_hbm, d1_hbm, table_hbm, out_hbm, idx0_v, idx1_v, rows_v, out_v,
          sem0, sem1):
    wid = lax.axis_index("s") * NC + lax.axis_index("c")
    base_row = wid * BPW
    sems = (sem0, sem1)

    def fire(g, b):
        row0 = base_row + g * G
        pltpu.sync_copy(d0_hbm.at[pl.ds(row0, G)], idx0_v.at[b])
        pltpu.sync_copy(d1_hbm.at[pl.ds(row0, G)], idx1_v.at[b])
        for r in range(G):
            pltpu.async_copy(
                table_hbm.at[idx0_v.at[b, r]],
                rows_v.at[b, pl.ds(r * L, C0)],
                sems[b],
            )
            pltpu.async_copy(
                table_hbm.at[idx1_v.at[b, r, pl.ds(128 - C1, C1)]],
                rows_v.at[b, pl.ds(r * L + C0, C1)],
                sems[b],
            )

    def drain(b):
        pltpu.make_async_copy(
            table_hbm.at[pl.ds(0, G * L)], rows_v.at[b], sems[b]
        ).wait()

    def accum(g, b):
        for r in range(G):
            e0 = r * L

            def rbody(i, accs):
                a0, a1 = accs
                e = e0 + i * UN
                for k in range(UN):
                    a0 = a0 + rows_v[b, e + k, 0:16]
                    a1 = a1 + rows_v[b, e + k, 16:32]
                return a0, a1

            z = jnp.zeros((16,), jnp.float32)
            a0, a1 = lax.fori_loop(0, L // UN, rbody, (z, z))
            out_v[b, r, 0:16] = a0
            out_v[b, r, 16:32] = a1
        pltpu.sync_copy(out_v.at[b], out_hbm.at[pl.ds(base_row + g * G, G)])

    fire(0, 0)

    @pl.loop(0, NG - 2, step=2)
    def _(g):
        fire(g + 1, 1)
        drain(0)
        accum(g, 0)
        fire(g + 2, 0)
        drain(1)
        accum(g + 1, 1)

    fire(NG - 1, 1)
    drain(0)
    accum(NG - 2, 0)
    drain(1)
    accum(NG - 1, 1)


def kernel(data, lengths, table):
    del lengths
    d0 = lax.slice(data, (0, 0), (B, C0))
    d1 = lax.slice(data, (0, L - 128), (B, L))
    table_c = _repack_table(table)
    mesh = plsc.VectorSubcoreMesh(core_axis_name="c", subcore_axis_name="s")
    f = pl.kernel(
        _body,
        out_type=jax.ShapeDtypeStruct((B, D), jnp.float32),
        mesh=mesh,
        scratch_types=[
            pltpu.VMEM((2, G, 128), jnp.int32),
            pltpu.VMEM((2, G, 128), jnp.int32),
            pltpu.VMEM((2, G * L, D), jnp.float32),
            pltpu.VMEM((2, G, D), jnp.float32),
            pltpu.SemaphoreType.DMA,
            pltpu.SemaphoreType.DMA,
        ],
        compiler_params=pltpu.CompilerParams(use_tc_tiling_on_sc=False),
    )
    return f(d0, d1, table_c)


# speedup vs baseline: 4.2718x; 1.0070x over previous
import jax
import jax.numpy as jnp
from jax import lax
from jax.experimental import pallas as pl
from jax.experimental.pallas import tpu as pltpu, tpu_sc as plsc

NC, NS = 2, 16
NW = NC * NS
B, L, D = 16384, 200, 32
V = 1000000
BPW = B // NW
G = 8
NG = BPW // G
UN = 8
C0 = 128
C1 = L - C0
BK = 8192


NBLK = 128
NVB = (V + BK - 1) // BK
BR = B // NBLK


def _tbody(t_ref, dT_ref, out_ref, d0_ref, d1_ref):
    full = t_ref[...].T
    g = full.reshape(BK // 4, 4, D)
    out_ref[...] = jnp.concatenate([g[:, a, :] for a in range(4)], axis=1)
    d0_ref[...] = dT_ref[0:C0, :].T
    d1_ref[...] = dT_ref[L - 128:L, :].T


def _repack(table, data):
    tT = table.T
    dT = data.T
    packed, d0, d1 = pl.pallas_call(
        _tbody,
        grid=(NBLK,),
        in_specs=[
            pl.BlockSpec((D, BK), lambda i: (0, jnp.minimum(i, NVB - 1))),
            pl.BlockSpec((L, BR), lambda i: (0, i)),
        ],
        out_specs=[
            pl.BlockSpec((BK // 4, 128), lambda i: (jnp.minimum(i, NVB - 1), 0)),
            pl.BlockSpec((BR, C0), lambda i: (i, 0)),
            pl.BlockSpec((BR, 128), lambda i: (i, 0)),
        ],
        out_shape=[
            jax.ShapeDtypeStruct((V // 4, 128), jnp.float32),
            jax.ShapeDtypeStruct((B, C0), jnp.int32),
            jax.ShapeDtypeStruct((B, 128), jnp.int32),
        ],
    )(tT, dT)
    return packed.reshape(V, D), d0, d1


def _body(d0_hbm, d1_hbm, table_hbm, out_hbm, idx0_v, idx1_v, rows_v, out_v,
          sem0, sem1):
    wid = lax.axis_index("s") * NC + lax.axis_index("c")
    base_row = wid * BPW
    sems = (sem0, sem1)

    def fire(g, b):
        row0 = base_row + g * G
        pltpu.sync_copy(d0_hbm.at[pl.ds(row0, G)], idx0_v.at[b])
        pltpu.sync_copy(d1_hbm.at[pl.ds(row0, G)], idx1_v.at[b])
        for r in range(G):
            pltpu.async_copy(
                table_hbm.at[idx0_v.at[b, r]],
                rows_v.at[b, pl.ds(r * L, C0)],
                sems[b],
            )
            pltpu.async_copy(
                table_hbm.at[idx1_v.at[b, r, pl.ds(128 - C1, C1)]],
                rows_v.at[b, pl.ds(r * L + C0, C1)],
                sems[b],
            )

    def drain(b):
        pltpu.make_async_copy(
            table_hbm.at[pl.ds(0, G * L)], rows_v.at[b], sems[b]
        ).wait()

    def accum(g, b):
        for r in range(G):
            e0 = r * L

            def rbody(i, accs):
                a0, a1 = accs
                e = e0 + i * UN
                for k in range(UN):
                    a0 = a0 + rows_v[b, e + k, 0:16]
                    a1 = a1 + rows_v[b, e + k, 16:32]
                return a0, a1

            z = jnp.zeros((16,), jnp.float32)
            a0, a1 = lax.fori_loop(0, L // UN, rbody, (z, z))
            out_v[b, r, 0:16] = a0
            out_v[b, r, 16:32] = a1
        pltpu.sync_copy(out_v.at[b], out_hbm.at[pl.ds(base_row + g * G, G)])

    fire(0, 0)

    @pl.loop(0, NG - 2, step=2)
    def _(g):
        fire(g + 1, 1)
        drain(0)
        accum(g, 0)
        fire(g + 2, 0)
        drain(1)
        accum(g + 1, 1)

    fire(NG - 1, 1)
    drain(0)
    accum(NG - 2, 0)
    drain(1)
    accum(NG - 1, 1)


def kernel(data, lengths, table):
    del lengths
    table_c, d0, d1 = _repack(table, data)
    mesh = plsc.VectorSubcoreMesh(core_axis_name="c", subcore_axis_name="s")
    f = pl.kernel(
        _body,
        out_type=jax.ShapeDtypeStruct((B, D), jnp.float32),
        mesh=mesh,
        scratch_types=[
            pltpu.VMEM((2, G, 128), jnp.int32),
            pltpu.VMEM((2, G, 128), jnp.int32),
            pltpu.VMEM((2, G * L, D), jnp.float32),
            pltpu.VMEM((2, G, D), jnp.float32),
            pltpu.SemaphoreType.DMA,
            pltpu.SemaphoreType.DMA,
        ],
        compiler_params=pltpu.CompilerParams(use_tc_tiling_on_sc=False),
    )
    return f(d0, d1, table_c)
